# baseline (speedup 1.0000x reference)
import functools

import jax
import jax.numpy as jnp
from jax import lax
from jax.experimental import pallas as pl
from jax.experimental.pallas import tpu as pltpu
import jax.experimental.pallas.tpu_sc as plsc

N_NODES = 50000
N_EDGES = 1600000
NC = 2
NS = 16
NW = NC * NS
N_PAD = 50176
R_TILE = N_PAD // NS
F3W = N_PAD * 3
FROWS = F3W // 128
CROWS = N_PAD // 128
CHUNK = 2000
PER_W = N_EDGES // NW
PT = N_PAD // NW
H2W = N_PAD * 16

_mesh = plsc.VectorSubcoreMesh(core_axis_name="c", subcore_axis_name="s")
_sc_params = pltpu.CompilerParams(use_tc_tiling_on_sc=False)


@functools.partial(
    pl.kernel,
    out_type=jax.ShapeDtypeStruct((NC, N_PAD, 3), jnp.float32),
    mesh=_mesh,
    compiler_params=_sc_params,
    scratch_types=[
        pltpu.VMEM((CHUNK,), jnp.int32),
        pltpu.VMEM((CHUNK, 3), jnp.float32),
        pltpu.VMEM((R_TILE, 3), jnp.float32),
        pltpu.VMEM_SHARED((N_PAD, 3), jnp.float32),
        pltpu.SemaphoreType.DMA,
    ],
)
def _sc_count(dst_hbm, ones_hbm, zeros_hbm, out_hbm,
              dst_v, ones_v, buf_v, acc_sh, sem):
    cid = lax.axis_index("c")
    sid = lax.axis_index("s")
    wid = cid * NS + sid
    pltpu.sync_copy(zeros_hbm, buf_v)
    pltpu.sync_copy(buf_v, acc_sh.at[pl.ds(sid * R_TILE, R_TILE)])
    pltpu.sync_copy(ones_hbm, ones_v)
    plsc.subcore_barrier()
    base = wid * PER_W

    def body(i, carry):
        off = base + i * CHUNK
        pltpu.sync_copy(dst_hbm.at[pl.ds(off, CHUNK)], dst_v)
        pltpu.sync_copy(ones_v, acc_sh.at[dst_v], add=True)
        return carry

    lax.fori_loop(0, PER_W // CHUNK, body, 0)
    plsc.subcore_barrier()
    pltpu.sync_copy(acc_sh.at[pl.ds(sid * R_TILE, R_TILE)], buf_v)
    pltpu.sync_copy(buf_v, out_hbm.at[cid, pl.ds(sid * R_TILE, R_TILE)])


@functools.partial(
    pl.kernel,
    out_type=jax.ShapeDtypeStruct((NC, N_PAD, 1), jnp.float32),
    mesh=_mesh,
    compiler_params=_sc_params,
    scratch_types=[
        pltpu.VMEM((CHUNK,), jnp.int32),
        pltpu.VMEM((CHUNK, 1), jnp.float32),
        pltpu.VMEM((R_TILE, 1), jnp.float32),
        pltpu.VMEM_SHARED((N_PAD, 1), jnp.float32),
        pltpu.SemaphoreType.DMA,
    ],
)
def _sc_count1(dst_hbm, ones_hbm, zeros_hbm, out_hbm,
               dst_v, ones_v, buf_v, acc_sh, sem):
    cid = lax.axis_index("c")
    sid = lax.axis_index("s")
    wid = cid * NS + sid
    pltpu.sync_copy(zeros_hbm, buf_v)
    pltpu.sync_copy(buf_v, acc_sh.at[pl.ds(sid * R_TILE, R_TILE)])
    pltpu.sync_copy(ones_hbm, ones_v)
    plsc.subcore_barrier()
    base = wid * PER_W

    def body(i, carry):
        off = base + i * CHUNK
        pltpu.sync_copy(dst_hbm.at[pl.ds(off, CHUNK)], dst_v)
        pltpu.sync_copy(ones_v, acc_sh.at[dst_v], add=True)
        return carry

    lax.fori_loop(0, PER_W // CHUNK, body, 0)
    plsc.subcore_barrier()
    pltpu.sync_copy(acc_sh.at[pl.ds(sid * R_TILE, R_TILE)], buf_v)
    pltpu.sync_copy(buf_v, out_hbm.at[cid, pl.ds(sid * R_TILE, R_TILE)])


@functools.partial(
    pl.kernel,
    out_type=jax.ShapeDtypeStruct((NC, N_PAD, 3), jnp.float32),
    mesh=_mesh,
    compiler_params=_sc_params,
    scratch_types=[
        pltpu.VMEM((CHUNK,), jnp.int32),
        pltpu.VMEM((CHUNK,), jnp.int32),
        pltpu.VMEM((CHUNK, 3), jnp.float32),
        pltpu.VMEM((R_TILE, 3), jnp.float32),
        pltpu.VMEM_SHARED((N_PAD, 3), jnp.float32),
        pltpu.SemaphoreType.DMA,
    ],
)
def _sc_scatter(g_hbm, src_hbm, dst_hbm, zeros_hbm, out_hbm,
                src_v, dst_v, rows_v, buf_v, acc_sh, sem):
    cid = lax.axis_index("c")
    sid = lax.axis_index("s")
    wid = cid * NS + sid
    pltpu.sync_copy(zeros_hbm, buf_v)
    pltpu.sync_copy(buf_v, acc_sh.at[pl.ds(sid * R_TILE, R_TILE)])
    plsc.subcore_barrier()
    base = wid * PER_W

    def body(i, carry):
        off = base + i * CHUNK
        pltpu.sync_copy(src_hbm.at[pl.ds(off, CHUNK)], src_v)
        pltpu.async_copy(g_hbm.at[src_v], rows_v, sem).wait()
        pltpu.sync_copy(dst_hbm.at[pl.ds(off, CHUNK)], dst_v)
        pltpu.sync_copy(rows_v, acc_sh.at[dst_v], add=True)
        return carry

    lax.fori_loop(0, PER_W // CHUNK, body, 0)
    plsc.subcore_barrier()
    pltpu.sync_copy(acc_sh.at[pl.ds(sid * R_TILE, R_TILE)], buf_v)
    pltpu.sync_copy(buf_v, out_hbm.at[cid, pl.ds(sid * R_TILE, R_TILE)])


@functools.partial(
    pl.kernel,
    out_type=jax.ShapeDtypeStruct((H2W,), jnp.float32),
    mesh=_mesh,
    compiler_params=pltpu.CompilerParams(use_tc_tiling_on_sc=False,
                                         needs_layout_passes=False),
    scratch_types=[
        pltpu.VMEM((PT * 3,), jnp.float32),
        pltpu.VMEM((PT * 3,), jnp.float32),
        pltpu.VMEM((PT * 3,), jnp.float32),
        pltpu.VMEM((PT * 3,), jnp.float32),
        pltpu.VMEM((64,), jnp.float32),
        pltpu.VMEM((PT * 16,), jnp.float32),
    ],
)
def _sc_h2(s2_hbm, g2_hbm, dis3_hbm, w2_hbm, out_hbm,
           s2a_v, s2b_v, g2_v, p_v, w2_v, h2_v):
    cid = lax.axis_index("c")
    sid = lax.axis_index("s")
    wid = cid * NS + sid
    woff = wid * (PT * 3)
    pltpu.sync_copy(s2_hbm.at[0, pl.ds(woff, PT * 3)], s2a_v)
    pltpu.sync_copy(s2_hbm.at[1, pl.ds(woff, PT * 3)], s2b_v)
    pltpu.sync_copy(g2_hbm.at[pl.ds(woff, PT * 3)], g2_v)
    pltpu.sync_copy(dis3_hbm.at[pl.ds(woff, PT * 3)], p_v)
    pltpu.sync_copy(w2_hbm, w2_v)

    def pbody(i, carry):
        sl = pl.ds(i * 16, 16)
        p_v[sl] = p_v[sl] * (s2a_v[sl] + s2b_v[sl] + g2_v[sl])
        return carry

    lax.fori_loop(0, (PT * 3) // 16, pbody, 0)

    iota = lax.iota(jnp.int32, 16)
    w2c0 = w2_v[pl.ds(0, 16)]
    w2c1 = w2_v[pl.ds(16, 16)]
    w2c2 = w2_v[pl.ds(32, 16)]
    b2v = w2_v[pl.ds(48, 16)]

    def gbody(g, carry):
        base3 = g * 48
        p0 = plsc.load_gather(p_v, [iota * 3 + base3])
        p1 = plsc.load_gather(p_v, [iota * 3 + (base3 + 1)])
        p2 = plsc.load_gather(p_v, [iota * 3 + (base3 + 2)])
        out_base = g * 256
        for e in range(16):
            a = jnp.sum(jnp.where(iota == e, w2c0, 0.0))
            b = jnp.sum(jnp.where(iota == e, w2c1, 0.0))
            c = jnp.sum(jnp.where(iota == e, w2c2, 0.0))
            d = jnp.sum(jnp.where(iota == e, b2v, 0.0))
            v = p0 * a + p1 * b + p2 * c + d
            plsc.store_scatter(h2_v, [iota * 16 + (out_base + e)], v)
        return carry

    lax.fori_loop(0, PT // 16, gbody, 0)
    pltpu.sync_copy(h2_v, out_hbm.at[pl.ds(wid * (PT * 16), PT * 16)])


def _word_index():
    row = lax.broadcasted_iota(jnp.int32, (FROWS, 128), 0)
    lane = lax.broadcasted_iota(jnp.int32, (FROWS, 128), 1)
    return row * 128 + lane


def _tc_dis_body(c0_ref, c1_ref, dis_ref):
    dis_ref[...] = lax.rsqrt(c0_ref[0] + c1_ref[0] + 1.0)


CROWS_ = N_PAD // 128

_tc_dis = pl.pallas_call(
    _tc_dis_body,
    grid=(1,),
    in_specs=[
        pl.BlockSpec((1, CROWS_, 128), lambda i: (0, 0, 0)),
        pl.BlockSpec((1, CROWS_, 128), lambda i: (1, 0, 0)),
    ],
    out_specs=pl.BlockSpec((CROWS_, 128), lambda i: (0, 0)),
    out_shape=jax.ShapeDtypeStruct((CROWS_, 128), jnp.float32),
)


def _tc_prep_body(dis3_ref, x0_ref, x1_ref, x2_ref, x3_ref, x4_ref,
                  w1_ref, g1_ref):
    dis3 = dis3_ref[...]
    r = _word_index() % 3
    m0 = jnp.where(r == 0, 1.0, 0.0)
    m1 = jnp.where(r == 1, 1.0, 0.0)
    m2 = jnp.where(r == 2, 1.0, 0.0)
    h = (m2 * w1_ref[2, 0] * x0_ref[...]
         + (m1 * w1_ref[1, 0] + m2 * w1_ref[2, 1]) * x1_ref[...]
         + (m0 * w1_ref[0, 0] + m1 * w1_ref[1, 1] + m2 * w1_ref[2, 2])
         * x2_ref[...]
         + (m0 * w1_ref[0, 1] + m1 * w1_ref[1, 2]) * x3_ref[...]
         + m0 * w1_ref[0, 2] * x4_ref[...])
    g1_ref[...] = h * dis3


_tc_prep = pl.pallas_call(
    _tc_prep_body,
    grid=(1,),
    in_specs=[
        pl.BlockSpec((FROWS, 128), lambda i: (0, 0)),
        pl.BlockSpec((FROWS, 128), lambda i: (0, 0)),
        pl.BlockSpec((FROWS, 128), lambda i: (0, 0)),
        pl.BlockSpec((FROWS, 128), lambda i: (0, 0)),
        pl.BlockSpec((FROWS, 128), lambda i: (0, 0)),
        pl.BlockSpec((FROWS, 128), lambda i: (0, 0)),
        pl.BlockSpec((3, 3), lambda i: (0, 0)),
    ],
    out_specs=pl.BlockSpec((FROWS, 128), lambda i: (0, 0)),
    out_shape=jax.ShapeDtypeStruct((FROWS, 128), jnp.float32),
)


def _tc_mid_body(s1a_ref, s1b_ref, g1_ref, dis3_ref, b1_ref, g2_ref):
    r = _word_index() % 3
    b1p = (jnp.where(r == 0, b1_ref[0, 0], 0.0)
           + jnp.where(r == 1, b1_ref[0, 1], 0.0)
           + jnp.where(r == 2, b1_ref[0, 2], 0.0))
    dis3 = dis3_ref[...]
    pre = dis3 * (s1a_ref[0] + s1b_ref[0] + g1_ref[...]) + b1p
    h1 = jnp.where(pre >= 0, pre, 0.1 * pre)
    g2_ref[...] = h1 * dis3


_tc_mid = pl.pallas_call(
    _tc_mid_body,
    grid=(1,),
    in_specs=[
        pl.BlockSpec((1, FROWS, 128), lambda i: (0, 0, 0)),
        pl.BlockSpec((1, FROWS, 128), lambda i: (1, 0, 0)),
        pl.BlockSpec((FROWS, 128), lambda i: (0, 0)),
        pl.BlockSpec((FROWS, 128), lambda i: (0, 0)),
        pl.BlockSpec((1, 3), lambda i: (0, 0)),
    ],
    out_specs=pl.BlockSpec((FROWS, 128), lambda i: (0, 0)),
    out_shape=jax.ShapeDtypeStruct((FROWS, 128), jnp.float32),
)


def _tc_final_body(h2_ref, w3_ref, b3_ref, out_ref):
    out = lax.dot_general(h2_ref[...], w3_ref[...], (((1,), (1,)), ((), ())),
                          preferred_element_type=jnp.float32)
    out_ref[...] = out + b3_ref[...]


_tc_final = pl.pallas_call(
    _tc_final_body,
    out_shape=jax.ShapeDtypeStruct((50, 128), jnp.float32),
)


def kernel(nodes, edges, W1, b1, W2, b2, W3, b3):
    src = edges[0, 0].astype(jnp.int32)
    dst = edges[0, 1].astype(jnp.int32)
    ones = jnp.ones((CHUNK, 1), jnp.float32)
    zeros1 = jnp.zeros((R_TILE, 1), jnp.float32)
    zeros3 = jnp.zeros((R_TILE, 3), jnp.float32)

    xe = jnp.concatenate([
        jnp.zeros((2,), jnp.float32),
        nodes.reshape(-1),
        jnp.zeros((F3W - 3 * N_NODES + 2,), jnp.float32),
    ])
    xs = [lax.slice(xe, (s,), (s + F3W,)).reshape(FROWS, 128)
          for s in range(5)]

    cnt = _sc_count1(dst, ones, zeros1)
    dis_p = _tc_dis(cnt.reshape(NC, CROWS_, 128), cnt.reshape(NC, CROWS_, 128))
    dis3f = jnp.broadcast_to(dis_p.reshape(N_PAD, 1),
                             (N_PAD, 3)).reshape(FROWS, 128)
    g1f = _tc_prep(dis3f, xs[0], xs[1], xs[2], xs[3], xs[4], W1)

    s1 = _sc_scatter(g1f.reshape(N_PAD, 3), src, dst, zeros3)
    s1f = s1.reshape(NC, FROWS, 128)
    g2f = _tc_mid(s1f, s1f, g1f, dis3f, b1.reshape(1, 3))

    s2 = _sc_scatter(g2f.reshape(N_PAD, 3), src, dst, zeros3)

    w2b2 = jnp.concatenate([W2[:, 0], W2[:, 1], W2[:, 2], b2])
    h2f = _sc_h2(s2.reshape(NC, F3W), g2f.reshape(F3W),
                 dis3f.reshape(F3W), w2b2)
    h2 = h2f[:N_NODES * 16].reshape(50, 16000)
    out = _tc_final(h2, W3, b3.reshape(1, 128))
    return out
